# baseline (speedup 1.0000x reference)
import functools

import jax
import jax.numpy as jnp
from jax import lax
from jax.experimental import pallas as pl
from jax.experimental.pallas import tpu as pltpu
from jax.experimental.pallas import tpu_sc as plsc

NSEL = 100
NBP = 17
B = 16
N = 8192
C = 2
NF = N * C
L = 16
NPAD = 112
GBUF = 128
KRES_PAD = 5760
DEPTH = 8


def _tc_thresh_body(logits_ref, prob_ref, t_ref):
    x = logits_ref[...]
    p = jax.nn.sigmoid(x)
    prob_ref[...] = p
    bits = lax.bitcast_convert_type(p, jnp.int32)

    def body(_, carry):
        lo, hi = carry
        mid = lax.shift_right_arithmetic(lo + hi, 1)
        cnt = jnp.sum((bits > mid).astype(jnp.int32), axis=1, keepdims=True)
        pred = cnt < NSEL
        return jnp.where(pred, lo, mid), jnp.where(pred, mid, hi)

    lo0 = jnp.full((B, 1), -1, jnp.int32)
    hi0 = jnp.full((B, 1), 0x3F800001, jnp.int32)
    _, hi = lax.fori_loop(0, 31, body, (lo0, hi0))
    t_ref[...] = jnp.broadcast_to(lax.bitcast_convert_type(hi, jnp.float32), (B, 128))


def _sc_body(probs_hbm, tval_hbm, ts_hbm, box2, kp2, pose2, beta2, cam2, kp3d2,
             score_o, label_o, boxf_o, kresf_o, pose_o, beta_o, cam_o, kp3d_o,
             probs_v, t_v, ts_v, gt_val, gt_idx, eq_idx, ord_val, ord_idx,
             grow_v, box_rows, kp_rows, pose_rows, beta_rows,
             cam_rows, kp3d_rows, box_out_v, kres_v, scores_v, labels_v, sem):
    cid = lax.axis_index("c")
    sid = lax.axis_index("s")
    b = sid

    @pl.when(cid == 0)
    def _work():
        iota = lax.iota(jnp.int32, L)
        pltpu.sync_copy(probs_hbm.at[b], probs_v)
        pltpu.sync_copy(tval_hbm, t_v.at[pl.ds(0, B)])
        pltpu.sync_copy(ts_hbm, ts_v.at[pl.ds(0, 2 * B)])

        bs = jnp.full((L,), b, jnp.int32)
        tb = plsc.load_gather(t_v, [bs])
        imgh = plsc.load_gather(ts_v, [2 * bs])
        imgw = plsc.load_gather(ts_v, [2 * bs + 1])

        zero_i = jnp.zeros((L,), jnp.int32)
        for k in range(GBUF // L):
            gt_val[pl.ds(k * L, L)] = jnp.full((L,), -1.0, jnp.float32)
            gt_idx[pl.ds(k * L, L)] = zero_i
            eq_idx[pl.ds(k * L, L)] = zero_i

        def scan_body(i, carry):
            ngt, neq = carry
            v = probs_v[pl.ds(i * L, L)]
            m_gt = v > tb
            m_eq = v == tb
            n_any = jnp.sum(jnp.where(m_gt | m_eq, 1, 0))

            def app(c):
                ngt, neq = c
                idxv = i * L + iota
                pre = plsc.cumsum(jnp.where(m_eq, 1, 0))
                m_aeq = m_eq & ((neq + pre) <= NSEL)
                plsc.store_compressed(gt_val.at[pl.ds(ngt, L)], v, mask=m_gt)
                plsc.store_compressed(gt_idx.at[pl.ds(ngt, L)], idxv, mask=m_gt)
                plsc.store_compressed(eq_idx.at[pl.ds(neq, L)], idxv, mask=m_aeq)
                return (ngt + jnp.sum(jnp.where(m_gt, 1, 0)),
                        neq + jnp.sum(jnp.where(m_aeq, 1, 0)))

            return lax.cond(n_any > 0, app, lambda c: c, (ngt, neq))

        ngt, neq = lax.fori_loop(0, NF // L, scan_body,
                                 (jnp.zeros((), jnp.int32), jnp.zeros((), jnp.int32)))

        for k in range(NPAD // L):
            pos = k * L + iota
            ep = jnp.maximum(pos - ngt, 0)
            ev = plsc.load_gather(eq_idx, [ep])
            ord_idx[pl.ds(k * L, L)] = jnp.where(pos >= ngt, ev, 0)
            ord_val[pl.ds(k * L, L)] = tb

        lane0 = iota == 0

        def rank_body(y, _):
            ys = jnp.full((L,), y, jnp.int32)
            yv = plsc.load_gather(gt_val, [ys])
            yi = plsc.load_gather(gt_idx, [ys])
            r = zero_i
            for k in range(GBUF // L):
                vk = gt_val[pl.ds(k * L, L)]
                ik = gt_idx[pl.ds(k * L, L)]
                beats = (vk > yv) | ((vk == yv) & (ik < yi))
                r = r + jnp.where(beats, 1, 0)
            rank = jnp.sum(r)
            rs = jnp.full((L,), rank, jnp.int32)
            plsc.store_scatter(ord_val, [rs], yv, mask=lane0)
            plsc.store_scatter(ord_idx, [rs], yi, mask=lane0)
            return 0

        lax.fori_loop(0, ngt, rank_body, 0)

        for k in range(GBUF // L):
            if k < NPAD // L:
                ov = ord_val[pl.ds(k * L, L)]
                oi = ord_idx[pl.ds(k * L, L)]
            else:
                ov = jnp.zeros((L,), jnp.float32)
                oi = zero_i
            scores_v[pl.ds(k * L, L)] = ov
            labels_v[pl.ds(k * L, L)] = lax.rem(oi, 2)
            grow_v[pl.ds(k * L, L)] = lax.div(oi, 2) + b * N

        def row_copies(y):
            r = grow_v[pl.ds(y, L)][0]
            return (
                pltpu.make_async_copy(box2.at[r], box_rows.at[y], sem),
                pltpu.make_async_copy(kp2.at[r], kp_rows.at[y], sem),
                pltpu.make_async_copy(pose2.at[r], pose_rows.at[y], sem),
                pltpu.make_async_copy(beta2.at[r], beta_rows.at[y], sem),
                pltpu.make_async_copy(cam2.at[r], cam_rows.at[y], sem),
                pltpu.make_async_copy(kp3d2.at[r], kp3d_rows.at[y], sem),
            )

        def dma_body(y, _):
            @pl.when(y < NSEL)
            def _issue():
                for cp in row_copies(y):
                    cp.start()

            @pl.when(y >= DEPTH)
            def _drain():
                for cp in row_copies(y - DEPTH):
                    cp.wait()

            return 0

        lax.fori_loop(0, NSEL + DEPTH, dma_body, 0)

        pltpu.sync_copy(scores_v, score_o.at[b])
        pltpu.sync_copy(labels_v, label_o.at[b])
        pltpu.sync_copy(pose_rows, pose_o.at[b])
        pltpu.sync_copy(beta_rows, beta_o.at[b])
        pltpu.sync_copy(cam_rows, cam_o.at[b])
        pltpu.sync_copy(kp3d_rows, kp3d_o.at[b])

        for k in range(4 * NPAD // L):
            q = k * L + iota
            s = lax.div(q, 4)
            comp = lax.rem(q, 4)
            c01 = lax.rem(comp, 2)
            cen = plsc.load_gather(box_rows, [s, c01])
            whv = plsc.load_gather(box_rows, [s, 2 + c01])
            sgn = jnp.where(comp < 2, -0.5, 0.5)
            scl = jnp.where(c01 == 0, imgw, imgh)
            box_out_v[pl.ds(k * L, L)] = (cen + sgn * whv) * scl
        pltpu.sync_copy(box_out_v, boxf_o.at[b])

        srcs = []
        scls = []
        one_f = jnp.ones((L,), jnp.float32)
        for j in range(4):
            p = j * L + iota
            r3 = lax.rem(p, 3)
            jj = lax.div(p, 3)
            src = jnp.where(r3 == 0, 2 * jj, jnp.where(r3 == 1, 2 * jj + 1, 34 + jj))
            srcs.append(jnp.where(p < 51, src, 0))
            scls.append(jnp.where(r3 == 0, imgw, jnp.where(r3 == 1, imgh, one_f)))
        mask3 = iota < 3

        def kres_body(s, _):
            ss = jnp.full((L,), s, jnp.int32)
            base = s * 51
            for j in range(3):
                vals = plsc.load_gather(kp_rows, [ss, srcs[j]])
                kres_v[pl.ds(base + j * L, L)] = vals * scls[j]
            vals = plsc.load_gather(kp_rows, [ss, srcs[3]])
            plsc.store_compressed(kres_v.at[pl.ds(base + 3 * L, L)],
                                  vals * scls[3], mask=mask3)
            return 0

        lax.fori_loop(0, NSEL, kres_body, 0)
        pltpu.sync_copy(kres_v, kresf_o.at[b])


def kernel(pred_logits, pred_boxes, pred_keypoints, pred_smpl_pose, pred_smpl_beta, pred_smpl_cam, pred_smpl_kp3d, target_sizes):
    flat = pred_logits.reshape(B, NF)
    probs, tvals = pl.pallas_call(
        _tc_thresh_body,
        out_shape=[
            jax.ShapeDtypeStruct((B, NF), jnp.float32),
            jax.ShapeDtypeStruct((B, 128), jnp.float32),
        ],
    )(flat)
    tval = tvals[:, 0]
    ts = target_sizes.astype(jnp.float32).reshape(2 * B)

    mesh = plsc.VectorSubcoreMesh(core_axis_name="c", subcore_axis_name="s")
    out_type = (
        jax.ShapeDtypeStruct((B, 128), jnp.float32),
        jax.ShapeDtypeStruct((B, 128), jnp.int32),
        jax.ShapeDtypeStruct((B, 4 * NPAD), jnp.float32),
        jax.ShapeDtypeStruct((B, KRES_PAD), jnp.float32),
        jax.ShapeDtypeStruct((B, NPAD, 216), jnp.float32),
        jax.ShapeDtypeStruct((B, NPAD, 10), jnp.float32),
        jax.ShapeDtypeStruct((B, NPAD, 3), jnp.float32),
        jax.ShapeDtypeStruct((B, NPAD, 51), jnp.float32),
    )
    scratch_types = [
        pltpu.VMEM((NF,), jnp.float32),
        pltpu.VMEM((128,), jnp.float32),
        pltpu.VMEM((128,), jnp.float32),
        pltpu.VMEM((GBUF,), jnp.float32),
        pltpu.VMEM((GBUF,), jnp.int32),
        pltpu.VMEM((GBUF,), jnp.int32),
        pltpu.VMEM((GBUF,), jnp.float32),
        pltpu.VMEM((GBUF,), jnp.int32),
        pltpu.VMEM((GBUF,), jnp.int32),
        pltpu.VMEM((NPAD, 4), jnp.float32),
        pltpu.VMEM((NPAD, 51), jnp.float32),
        pltpu.VMEM((NPAD, 216), jnp.float32),
        pltpu.VMEM((NPAD, 10), jnp.float32),
        pltpu.VMEM((NPAD, 3), jnp.float32),
        pltpu.VMEM((NPAD, 51), jnp.float32),
        pltpu.VMEM((4 * NPAD,), jnp.float32),
        pltpu.VMEM((KRES_PAD,), jnp.float32),
        pltpu.VMEM((128,), jnp.float32),
        pltpu.VMEM((128,), jnp.int32),
        pltpu.SemaphoreType.DMA,
    ]
    sc = functools.partial(
        pl.kernel, mesh=mesh, out_type=out_type, scratch_types=scratch_types,
        compiler_params=pltpu.CompilerParams(needs_layout_passes=False),
    )(_sc_body)
    score_p, label_p, boxf_p, kresf_p, pose_p, beta_p, cam_p, kp3d_p = sc(
        probs, tval, ts,
        pred_boxes.reshape(B * N, 4),
        pred_keypoints.reshape(B * N, NBP * 3),
        pred_smpl_pose.reshape(B * N, 216),
        pred_smpl_beta.reshape(B * N, 10),
        pred_smpl_cam.reshape(B * N, 3),
        pred_smpl_kp3d.reshape(B * N, NBP * 3),
    )

    scores = score_p[:, :NSEL]
    labels = label_p[:, :NSEL]
    boxes_out = boxf_p[:, :4 * NSEL].reshape(B, NSEL, 4)
    kres = kresf_p[:, :NPAD * 51].reshape(B, NPAD, 51)[:, :NSEL]
    smpl_pose = pose_p[:, :NSEL].reshape(B, NSEL, 24, 3, 3)
    smpl_beta = beta_p[:, :NSEL]
    smpl_cam = cam_p[:, :NSEL]
    smpl_kp3d = kp3d_p[:, :NSEL].reshape(B, NSEL, NBP, 3)
    return (scores, labels, boxes_out, kres, smpl_pose, smpl_beta, smpl_cam, smpl_kp3d)
